# baseline (speedup 1.0000x reference)
import jax
import jax.numpy as jnp
from jax.experimental import pallas as pl
from jax.experimental.pallas import tpu as pltpu

_K = 256
_CT = 256


def _fused_body(ic_ref, mask_ref, x_ref, o_ref, g_ref):
    j = pl.program_id(1)
    c = mask_ref.shape[2]
    ct = x_ref.shape[1]

    @pl.when(j == 0)
    def _compute_gate():
        m_col = mask_ref[0, 0, :].reshape(c, 1)
        m_row = mask_ref[0, 0, :].reshape(1, c)
        counts = jnp.sum((m_row <= m_col).astype(jnp.float32), axis=1,
                         keepdims=True)
        kth = jnp.min(jnp.where(counts >= _K, m_col, jnp.inf))
        thr = jnp.where(ic_ref[0, 0] > 0, kth, -jnp.inf)
        g_ref[:, :] = jnp.where(m_col <= thr, 0.0, m_col)

    o_ref[0] = x_ref[0] * g_ref[pl.ds(j * ct, ct), :]


def kernel(x, x_averaged, inactive_channels):
    b, c, l = x.shape
    mask = x_averaged.reshape(b, 1, c)
    ic = jnp.asarray(inactive_channels, jnp.int32).reshape(1, 1)
    ct = _CT if c % _CT == 0 else c

    out = pl.pallas_call(
        _fused_body,
        grid=(b, c // ct),
        in_specs=[
            pl.BlockSpec(memory_space=pltpu.SMEM),
            pl.BlockSpec((1, 1, c), lambda i, j: (i, 0, 0)),
            pl.BlockSpec((1, ct, l), lambda i, j: (i, j, 0)),
        ],
        out_specs=pl.BlockSpec((1, ct, l), lambda i, j: (i, j, 0)),
        out_shape=jax.ShapeDtypeStruct((b, c, l), x.dtype),
        scratch_shapes=[pltpu.VMEM((c, 1), jnp.float32)],
    )(ic, mask, x)
    return (out, 0.0)
